# baseline (speedup 1.0000x reference)
import functools

import jax
import jax.numpy as jnp
from jax import lax
from jax.experimental import pallas as pl
from jax.experimental.pallas import tpu as pltpu
from jax.experimental.pallas import tpu_sc as plsc

NC = 2
NS = 16
NW = NC * NS
L = 16

ADIM = 16
VDIM = 16
KDIM = 24
ODIM = ADIM + 3 * VDIM

C = 1000
CP = 1008
GROUPS = CP // L
BE = 2000


def _sc_body(table, idxp, inter, idx_v, gS, gD, og, sem):
    n_edges = inter.shape[0]
    ew = n_edges // NW
    nch = ew // C
    sid = lax.axis_index("s")
    wid = sid * NC + lax.axis_index("c")

    iota = lax.iota(jnp.int32, L)
    cfull = [jnp.full((L,), c, jnp.int32) for c in range(6)]
    kfull = [jnp.full((L,), k, jnp.int32) for k in range(18)]

    def chunk_body(i, carry):
        base = wid * ew + i * C
        r = wid * nch + i

        pltpu.sync_copy(idxp.at[pl.ds(r, 1)], idx_v)

        copies = [
            pltpu.async_copy(table.at[idx_v.at[0, pl.ds(0, CP)]], gS, sem),
            pltpu.async_copy(table.at[idx_v.at[0, pl.ds(CP, CP)]], gD, sem),
        ]
        for cp in copies:
            cp.wait()

        def group_body(g, carry2):
            rows = g * L + iota
            p = []
            for buf, off in ((gS, 0), (gD, 3)):
                for c in range(6):
                    col = plsc.load_gather(buf, [rows, cfull[c]])
                    p.append(col)
                    t2 = (off + (3 if c >= 3 else 0)) + (c % 3)
                    plsc.store_scatter(og, [rows, kfull[t2]], col)
            p0s = p[0:3]; p1s = p[3:6]; p0d = p[6:9]; p1d = p[9:12]

            nacc = [None] * 6
            for c in range(3):
                d0 = p0d[c] - p0s[c]
                d1 = p1d[c] - p1s[c]
                d2 = p1s[c] - p0s[c]
                d3 = p1d[c] - p0d[c]
                d4 = p1s[c] - p0d[c]
                d5 = p1d[c] - p0s[c]
                for k, dk in enumerate((d0, d1, d2, d3, d4, d5)):
                    sq = dk * dk
                    nacc[k] = sq if nacc[k] is None else nacc[k] + sq

            for k in range(6):
                x = jnp.maximum(nacc[k], jnp.float32(1e-12))
                iv = plsc.bitcast(x, jnp.int32)
                iv = jnp.int32(0x5F3759DF) - (iv >> 1)
                y = plsc.bitcast(iv, jnp.float32)
                xh = x * jnp.float32(0.5)
                y = y * (jnp.float32(1.5) - xh * y * y)
                y = y * (jnp.float32(1.5) - xh * y * y)
                plsc.store_scatter(og, [rows, kfull[12 + k]], x * y)
            return carry2

        lax.fori_loop(0, GROUPS, group_body, 0, unroll=False)

        pltpu.sync_copy(og.at[pl.ds(0, C)], inter.at[pl.ds(base, C)])
        return carry

    lax.fori_loop(0, nch, chunk_body, 0, unroll=False)


def _tc_body(lhs_ref, wh_ref, wl_ref, bias_ref, a_ref, v_ref):
    x = lhs_ref[...]
    xh = x.astype(jnp.bfloat16)
    xl = (x - xh.astype(jnp.float32)).astype(jnp.bfloat16)
    wh = wh_ref[...]
    wl = wl_ref[...]
    r = jnp.dot(xh, wh, preferred_element_type=jnp.float32)
    r = r + jnp.dot(xl, wh, preferred_element_type=jnp.float32)
    r = r + jnp.dot(xh, wl, preferred_element_type=jnp.float32)
    r = r + bias_ref[...]
    a_ref[...] = r[:, :ADIM]
    v_ref[...] = r[:, ADIM:ODIM]


def kernel(pos_0, pos_1, src, dst, Wa, ba, Wv):
    n_nodes = pos_0.shape[1]
    n_edges = src.shape[0]

    table = jnp.concatenate(
        [pos_0[0], pos_1[0], jnp.zeros((n_nodes, 2), jnp.float32)], axis=1)

    A = jnp.array(
        [[-1.0, 1.0, 0.0, 0.0],
         [0.0, 0.0, -1.0, 1.0],
         [-1.0, 0.0, 1.0, 0.0],
         [0.0, -1.0, 0.0, 1.0],
         [0.0, -1.0, 1.0, 0.0],
         [-1.0, 0.0, 0.0, 1.0]], dtype=jnp.float32)
    Wq = (Wv @ A) / 3.0

    Wfull = jnp.zeros((KDIM, ODIM), jnp.float32)
    jidx = jnp.arange(VDIM)
    for t in range(4):
        for c in range(3):
            Wfull = Wfull.at[t * 3 + c, ADIM + jidx * 3 + c].set(Wq[:, t])
    for k in range(6):
        Wfull = Wfull.at[12 + k, :ADIM].set(Wa[:, k])
    bias = jnp.zeros((1, ODIM), jnp.float32).at[0, :ADIM].set(ba)
    W_hi = Wfull.astype(jnp.bfloat16)
    W_lo = (Wfull - W_hi.astype(jnp.float32)).astype(jnp.bfloat16)

    nrows = n_edges // C
    zpad = jnp.zeros((nrows, CP - C), jnp.int32)
    idxp = jnp.concatenate(
        [src.reshape(nrows, C), zpad, dst.reshape(nrows, C), zpad], axis=1)

    mesh = plsc.VectorSubcoreMesh(
        core_axis_name="c", subcore_axis_name="s",
        num_cores=NC, num_subcores=NS)

    sc_run = pl.kernel(
        _sc_body,
        out_type=jax.ShapeDtypeStruct((n_edges, KDIM), jnp.float32),
        mesh=mesh,
        compiler_params=pltpu.CompilerParams(
            needs_layout_passes=False, use_tc_tiling_on_sc=False),
        scratch_types=[
            pltpu.VMEM((1, 2 * CP), jnp.int32),
            pltpu.VMEM((CP, 8), jnp.float32),
            pltpu.VMEM((CP, 8), jnp.float32),
            pltpu.VMEM((CP, KDIM), jnp.float32),
            pltpu.SemaphoreType.DMA,
        ],
    )
    inter = sc_run(table, idxp)

    a2d, v2d = pl.pallas_call(
        _tc_body,
        out_shape=(
            jax.ShapeDtypeStruct((n_edges, ADIM), jnp.float32),
            jax.ShapeDtypeStruct((n_edges, 3 * VDIM), jnp.float32),
        ),
        grid=(n_edges // BE,),
        in_specs=[
            pl.BlockSpec((BE, KDIM), lambda i: (i, 0)),
            pl.BlockSpec((KDIM, ODIM), lambda i: (0, 0)),
            pl.BlockSpec((KDIM, ODIM), lambda i: (0, 0)),
            pl.BlockSpec((1, ODIM), lambda i: (0, 0)),
        ],
        out_specs=(
            pl.BlockSpec((BE, ADIM), lambda i: (i, 0)),
            pl.BlockSpec((BE, 3 * VDIM), lambda i: (i, 0)),
        ),
    )(inter, W_hi, W_lo, bias)

    return (a2d.reshape(1, n_edges, ADIM),
            v2d.reshape(1, n_edges, VDIM, 3))


# speedup vs baseline: 1.0435x; 1.0435x over previous
import functools

import jax
import jax.numpy as jnp
from jax import lax
from jax.experimental import pallas as pl
from jax.experimental.pallas import tpu as pltpu
from jax.experimental.pallas import tpu_sc as plsc

NC = 2
NS = 16
NW = NC * NS
L = 16

ADIM = 16
VDIM = 16

C = 1000
CP = 1008
GROUPS = CP // L
GSUB = 112
NSUB = CP // GSUB


def _body(table, idxp, wa_b, ba_b, wq_b, a_out, v_out,
          idx0, idx1, gS0, gD0, gS1, gD1, oa, ov, wa_v, ba_v, wq_v,
          sem0, sem1, semo):
    n_edges = a_out.shape[0]
    n_nodes = table.shape[0]
    ew = n_edges // NW
    nch = ew // C
    sid = lax.axis_index("s")
    wid = sid * NC + lax.axis_index("c")

    pltpu.sync_copy(wa_b, wa_v)
    pltpu.sync_copy(ba_b, ba_v)
    pltpu.sync_copy(wq_b, wq_v)

    iota = lax.iota(jnp.int32, L)
    cfull = [jnp.full((L,), c, jnp.int32) for c in range(6)]
    afull = [jnp.full((L,), j, jnp.int32) for j in range(ADIM)]
    vfull = [jnp.full((L,), j, jnp.int32) for j in range(3 * VDIM)]

    bufs = ((idx0, gS0, gD0, sem0), (idx1, gS1, gD1, sem1))

    def issue(r, b):
        idxb, gSb, gDb, semb = bufs[b]
        pltpu.sync_copy(idxp.at[pl.ds(r, 1)], idxb)
        pltpu.async_copy(table.at[idxb.at[0, pl.ds(0, CP)]], gSb, semb)
        pltpu.async_copy(table.at[idxb.at[0, pl.ds(CP, CP)]], gDb, semb)

    def wait_gathers(b):
        idxb, gSb, gDb, semb = bufs[b]
        pltpu.make_async_copy(table.at[idxb.at[0, pl.ds(0, CP)]], gSb, semb).wait()
        pltpu.make_async_copy(table.at[idxb.at[0, pl.ds(CP, CP)]], gDb, semb).wait()

    def compute_chunk(i, b):
        base = wid * ew + i * C
        _, gS, gD, _ = bufs[b]

        def group_body(g, carry2):
            rows = g * L + iota
            p = []
            for buf in (gS, gD):
                for c in range(6):
                    p.append(plsc.load_gather(buf, [rows, cfull[c]]))
            p0s = p[0:3]; p1s = p[3:6]; p0d = p[6:9]; p1d = p[9:12]

            nacc = [None] * 6
            for c in range(3):
                d0 = p0d[c] - p0s[c]
                d1 = p1d[c] - p1s[c]
                d2 = p1s[c] - p0s[c]
                d3 = p1d[c] - p0d[c]
                d4 = p1s[c] - p0d[c]
                d5 = p1d[c] - p0s[c]
                for k, dk in enumerate((d0, d1, d2, d3, d4, d5)):
                    sq = dk * dk
                    nacc[k] = sq if nacc[k] is None else nacc[k] + sq

            norms = []
            for k in range(6):
                x = jnp.maximum(nacc[k], jnp.float32(1e-12))
                iv = plsc.bitcast(x, jnp.int32)
                iv = jnp.int32(0x5F3759DF) - (iv >> 1)
                y = plsc.bitcast(iv, jnp.float32)
                xh = x * jnp.float32(0.5)
                y = y * (jnp.float32(1.5) - xh * y * y)
                y = y * (jnp.float32(1.5) - xh * y * y)
                norms.append(x * y)

            for j in range(ADIM):
                acc = ba_v[j]
                for k in range(6):
                    acc = acc + norms[k] * wa_v[j * 6 + k]
                plsc.store_scatter(oa, [rows, afull[j]], acc)

            pt = (p0s, p0d, p1s, p1d)
            for j in range(VDIM):
                w = [wq_v[j * 4 + t] for t in range(4)]
                for c in range(3):
                    acc = w[0] * pt[0][c]
                    for t in range(1, 4):
                        acc = acc + w[t] * pt[t][c]
                    plsc.store_scatter(ov, [rows, vfull[j * 3 + c]], acc)
            return carry2

        lax.fori_loop(0, GROUPS, group_body, 0, unroll=False)

        outc = [
            pltpu.async_copy(oa.at[pl.ds(0, C)], a_out.at[pl.ds(base, C)], semo),
            pltpu.async_copy(ov.at[pl.ds(0, C)], v_out.at[pl.ds(base, C)], semo),
        ]
        for cp in outc:
            cp.wait()

    r0 = wid * nch
    issue(r0, 0)

    def pair_body(j, carry):
        i0 = 2 * j
        issue(r0 + i0 + 1, 1)
        wait_gathers(0)
        compute_chunk(i0, 0)
        issue(r0 + i0 + 2, 0)
        wait_gathers(1)
        compute_chunk(i0 + 1, 1)
        return carry

    lax.fori_loop(0, nch // 2, pair_body, 0, unroll=False)
    wait_gathers(0)
    compute_chunk(nch - 1, 0)


def kernel(pos_0, pos_1, src, dst, Wa, ba, Wv):
    n_nodes = pos_0.shape[1]
    n_edges = src.shape[0]

    table = jnp.concatenate(
        [pos_0[0], pos_1[0], jnp.zeros((n_nodes, 2), jnp.float32)], axis=1)

    A = jnp.array(
        [[-1.0, 1.0, 0.0, 0.0],
         [0.0, 0.0, -1.0, 1.0],
         [-1.0, 0.0, 1.0, 0.0],
         [0.0, -1.0, 0.0, 1.0],
         [0.0, -1.0, 1.0, 0.0],
         [-1.0, 0.0, 0.0, 1.0]], dtype=jnp.float32)
    Wq = (Wv @ A) / 3.0

    wa_b = jnp.broadcast_to(Wa.reshape(ADIM * 6, 1), (ADIM * 6, L))
    ba_b = jnp.broadcast_to(ba.reshape(ADIM, 1), (ADIM, L))
    wq_b = jnp.broadcast_to(Wq.reshape(VDIM * 4, 1), (VDIM * 4, L))

    nrows = n_edges // C
    zpad = jnp.zeros((nrows, CP - C), jnp.int32)
    idxp = jnp.concatenate(
        [src.reshape(nrows, C), zpad, dst.reshape(nrows, C), zpad], axis=1)

    mesh = plsc.VectorSubcoreMesh(
        core_axis_name="c", subcore_axis_name="s",
        num_cores=NC, num_subcores=NS)

    run = pl.kernel(
        _body,
        out_type=(
            jax.ShapeDtypeStruct((n_edges, ADIM), jnp.float32),
            jax.ShapeDtypeStruct((n_edges, 3 * VDIM), jnp.float32),
        ),
        mesh=mesh,
        compiler_params=pltpu.CompilerParams(
            needs_layout_passes=False, use_tc_tiling_on_sc=False),
        scratch_types=[
            pltpu.VMEM((1, 2 * CP), jnp.int32),
            pltpu.VMEM((1, 2 * CP), jnp.int32),
            pltpu.VMEM((CP, 8), jnp.float32),
            pltpu.VMEM((CP, 8), jnp.float32),
            pltpu.VMEM((CP, 8), jnp.float32),
            pltpu.VMEM((CP, 8), jnp.float32),
            pltpu.VMEM((CP, ADIM), jnp.float32),
            pltpu.VMEM((CP, 3 * VDIM), jnp.float32),
            pltpu.VMEM((ADIM * 6, L), jnp.float32),
            pltpu.VMEM((ADIM, L), jnp.float32),
            pltpu.VMEM((VDIM * 4, L), jnp.float32),
            pltpu.SemaphoreType.DMA,
            pltpu.SemaphoreType.DMA,
            pltpu.SemaphoreType.DMA,
        ],
    )
    a_out, v_flat = run(table, idxp,
                        wa_b.astype(jnp.float32), ba_b.astype(jnp.float32),
                        wq_b.astype(jnp.float32))
    return (a_out.reshape(1, n_edges, ADIM),
            v_flat.reshape(1, n_edges, VDIM, 3))


# speedup vs baseline: 1.0441x; 1.0006x over previous
import jax
import jax.numpy as jnp
from jax import lax
from jax.experimental import pallas as pl
from jax.experimental.pallas import tpu as pltpu
from jax.experimental.pallas import tpu_sc as plsc

NC = 2
NS = 16
NW = NC * NS
L = 16

ADIM = 16
VDIM = 16

C = 1000
CP = 1008
GROUPS = CP // L


def _body(table, idxp, wa_b, ba_b, wq_b, a_out, v_out,
          idx0, idx1, gS0, gD0, gS1, gD1, oa, ov, wa_v, ba_v, wq_v,
          sem0, sem1, semo):
    n_edges = a_out.shape[0]
    n_nodes = table.shape[0]
    ew = n_edges // NW
    nch = ew // C
    sid = lax.axis_index("s")
    wid = sid * NC + lax.axis_index("c")

    pltpu.sync_copy(wa_b, wa_v)
    pltpu.sync_copy(ba_b, ba_v)
    pltpu.sync_copy(wq_b, wq_v)

    iota = lax.iota(jnp.int32, L)
    cfull = [jnp.full((L,), c, jnp.int32) for c in range(6)]
    afull = [jnp.full((L,), j, jnp.int32) for j in range(ADIM)]
    vfull = [jnp.full((L,), j, jnp.int32) for j in range(3 * VDIM)]

    bufs = ((idx0, gS0, gD0, sem0), (idx1, gS1, gD1, sem1))

    def issue(r, b):
        idxb, gSb, gDb, semb = bufs[b]
        pltpu.sync_copy(idxp.at[pl.ds(r, 1)], idxb)
        pltpu.async_copy(table.at[idxb.at[0, pl.ds(0, CP)]], gSb, semb)
        pltpu.async_copy(table.at[idxb.at[0, pl.ds(CP, CP)]], gDb, semb)

    def wait_gathers(b):
        idxb, gSb, gDb, semb = bufs[b]
        pltpu.make_async_copy(table.at[idxb.at[0, pl.ds(0, CP)]], gSb, semb).wait()
        pltpu.make_async_copy(table.at[idxb.at[0, pl.ds(CP, CP)]], gDb, semb).wait()

    def compute_chunk(i, b):
        base = wid * ew + i * C
        _, gS, gD, _ = bufs[b]

        def group_body(g, carry2):
            rows = g * L + iota
            p = []
            for buf in (gS, gD):
                for c in range(6):
                    p.append(plsc.load_gather(buf, [rows, cfull[c]]))
            p0s = p[0:3]; p1s = p[3:6]; p0d = p[6:9]; p1d = p[9:12]

            nacc = [None] * 6
            for c in range(3):
                d0 = p0d[c] - p0s[c]
                d1 = p1d[c] - p1s[c]
                d2 = p1s[c] - p0s[c]
                d3 = p1d[c] - p0d[c]
                d4 = p1s[c] - p0d[c]
                d5 = p1d[c] - p0s[c]
                for k, dk in enumerate((d0, d1, d2, d3, d4, d5)):
                    sq = dk * dk
                    nacc[k] = sq if nacc[k] is None else nacc[k] + sq

            norms = []
            for k in range(6):
                x = jnp.maximum(nacc[k], jnp.float32(1e-12))
                iv = plsc.bitcast(x, jnp.int32)
                iv = jnp.int32(0x5F3759DF) - (iv >> 1)
                y = plsc.bitcast(iv, jnp.float32)
                xh = x * jnp.float32(0.5)
                y = y * (jnp.float32(1.5) - xh * y * y)
                y = y * (jnp.float32(1.5) - xh * y * y)
                norms.append(x * y)

            for j in range(ADIM):
                acc = ba_v[j]
                for k in range(6):
                    acc = acc + norms[k] * wa_v[j * 6 + k]
                plsc.store_scatter(oa, [rows, afull[j]], acc)

            pt = (p0s, p0d, p1s, p1d)
            for j in range(VDIM):
                w = [wq_v[j * 4 + t] for t in range(4)]
                for c in range(3):
                    acc = w[0] * pt[0][c]
                    for t in range(1, 4):
                        acc = acc + w[t] * pt[t][c]
                    plsc.store_scatter(ov, [rows, vfull[j * 3 + c]], acc)
            return carry2

        lax.fori_loop(0, GROUPS, group_body, 0, unroll=False)

        outc = [
            pltpu.async_copy(oa.at[pl.ds(0, C)], a_out.at[pl.ds(base, C)], semo),
            pltpu.async_copy(ov.at[pl.ds(0, C)], v_out.at[pl.ds(base, C)], semo),
        ]
        for cp in outc:
            cp.wait()

    r0 = wid * nch
    issue(r0, 0)

    def pair_body(j, carry):
        i0 = 2 * j
        issue(r0 + i0 + 1, 1)
        wait_gathers(0)
        compute_chunk(i0, 0)
        issue(r0 + i0 + 2, 0)
        wait_gathers(1)
        compute_chunk(i0 + 1, 1)
        return carry

    lax.fori_loop(0, nch // 2, pair_body, 0, unroll=False)
    wait_gathers(0)
    compute_chunk(nch - 1, 0)


def kernel(pos_0, pos_1, src, dst, Wa, ba, Wv):
    n_nodes = pos_0.shape[1]
    n_edges = src.shape[0]

    table = jnp.concatenate(
        [pos_0[0], pos_1[0], jnp.zeros((n_nodes, 2), jnp.float32)], axis=1)

    A = jnp.array(
        [[-1.0, 1.0, 0.0, 0.0],
         [0.0, 0.0, -1.0, 1.0],
         [-1.0, 0.0, 1.0, 0.0],
         [0.0, -1.0, 0.0, 1.0],
         [0.0, -1.0, 1.0, 0.0],
         [-1.0, 0.0, 0.0, 1.0]], dtype=jnp.float32)
    Wq = (Wv @ A) / 3.0

    wa_b = jnp.broadcast_to(Wa.reshape(ADIM * 6, 1), (ADIM * 6, L))
    ba_b = jnp.broadcast_to(ba.reshape(ADIM, 1), (ADIM, L))
    wq_b = jnp.broadcast_to(Wq.reshape(VDIM * 4, 1), (VDIM * 4, L))

    nrows = n_edges // C
    zpad = jnp.zeros((nrows, CP - C), jnp.int32)
    idxp = jnp.concatenate(
        [src.reshape(nrows, C), zpad, dst.reshape(nrows, C), zpad], axis=1)

    mesh = plsc.VectorSubcoreMesh(
        core_axis_name="c", subcore_axis_name="s",
        num_cores=NC, num_subcores=NS)

    run = pl.kernel(
        _body,
        out_type=(
            jax.ShapeDtypeStruct((n_edges, ADIM), jnp.float32),
            jax.ShapeDtypeStruct((n_edges, 3 * VDIM), jnp.float32),
        ),
        mesh=mesh,
        compiler_params=pltpu.CompilerParams(
            needs_layout_passes=False, use_tc_tiling_on_sc=False),
        scratch_types=[
            pltpu.VMEM((1, 2 * CP), jnp.int32),
            pltpu.VMEM((1, 2 * CP), jnp.int32),
            pltpu.VMEM((CP, 8), jnp.float32),
            pltpu.VMEM((CP, 8), jnp.float32),
            pltpu.VMEM((CP, 8), jnp.float32),
            pltpu.VMEM((CP, 8), jnp.float32),
            pltpu.VMEM((CP, ADIM), jnp.float32),
            pltpu.VMEM((CP, 3 * VDIM), jnp.float32),
            pltpu.VMEM((ADIM * 6, L), jnp.float32),
            pltpu.VMEM((ADIM, L), jnp.float32),
            pltpu.VMEM((VDIM * 4, L), jnp.float32),
            pltpu.SemaphoreType.DMA,
            pltpu.SemaphoreType.DMA,
            pltpu.SemaphoreType.DMA,
        ],
    )
    a_out, v_flat = run(table, idxp,
                        wa_b.astype(jnp.float32), ba_b.astype(jnp.float32),
                        wq_b.astype(jnp.float32))
    return (a_out.reshape(1, n_edges, ADIM),
            v_flat.reshape(1, n_edges, VDIM, 3))
